# baseline (speedup 1.0000x reference)
import jax
import jax.numpy as jnp
from jax import lax
from jax.experimental import pallas as pl
from jax.experimental.pallas import tpu as pltpu
from jax.experimental.pallas import tpu_sc as plsc

N_CH = 192
N_PTS = 7
BOUND = 2.7
N_REG = N_PTS - 1
ROW = 224 * 224
NROWS = 4 * N_CH
LANES = 16

S_SC = 256
RB = 8
SB = S_SC // RB

NW = 32
NBUF = 4
CPR = 4
CHUNK = ROW // CPR
LANES_2D = (392, 128)

_INV_LEN = float(N_REG) / (2.0 * BOUND)
_SHIFT = BOUND * _INV_LEN


def _take16(vec, idx):
  return vec.at[idx].get(mode="promise_in_bounds")


def _sc_body(x_hbm, pts_hbm, out_hbm, pts_row, inbufs, outbufs, isems, osems):
  wid = lax.axis_index("s") * 2 + lax.axis_index("c")
  rows_per_w = S_SC // NW
  chunks_per_w = rows_per_w * CPR
  base_chunk = wid * chunks_per_w

  def start_in(g, b):
    pltpu.async_copy(x_hbm.at[pl.ds((base_chunk + g) * CHUNK, CHUNK)],
                     inbufs[b], isems[b])

  def wait_in(b):
    pltpu.make_async_copy(x_hbm.at[pl.ds(0, CHUNK)], inbufs[b],
                          isems[b]).wait()

  def start_out(g, b):
    pltpu.async_copy(outbufs[b],
                     out_hbm.at[pl.ds((base_chunk + g) * CHUNK, CHUNK)],
                     osems[b])

  def wait_out(b):
    pltpu.make_async_copy(outbufs[b], out_hbm.at[pl.ds(0, CHUNK)],
                          osems[b]).wait()

  for b in range(NBUF):
    start_in(b, b)

  lanes = lax.iota(jnp.int32, LANES)
  shift_idx = jnp.minimum(lanes + 1, LANES - 1)
  lanes_f = lanes.astype(jnp.float32)

  def row_body(j, carry):
    row = wid * rows_per_w + j
    ch = lax.rem(row, N_CH)
    pltpu.sync_copy(pts_hbm.at[ch], pts_row)
    p = pts_row[...]
    d = _take16(p, shift_idx) - p
    a = p - lanes_f * d

    for bb in range(CPR):
      g = j * CPR + bb
      b = bb
      wait_in(b)

      @pl.when(g >= NBUF)
      def _():
        wait_out(b)

      @plsc.parallel_loop(0, CHUNK, step=LANES, unroll=16)
      def _(off):
        xv = inbufs[b][pl.ds(off, LANES)]
        xn = xv * _INV_LEN + _SHIFT
        ri = jnp.minimum(jnp.maximum(xn.astype(jnp.int32), 0), N_REG - 1)
        outbufs[b][pl.ds(off, LANES)] = (
            _take16(a, ri) + xn * _take16(d, ri))

      start_out(g, b)

      @pl.when(g < chunks_per_w - NBUF)
      def _():
        start_in(g + NBUF, b)

    return carry

  lax.fori_loop(0, rows_per_w, row_body, 0)
  for b in range(NBUF):
    wait_out(b)


def _pwlu_sc(x_flat, pts_pad):
  mesh = plsc.VectorSubcoreMesh(core_axis_name="c", subcore_axis_name="s")
  return pl.kernel(
      _sc_body,
      out_type=jax.ShapeDtypeStruct((NROWS * ROW,), jnp.float32),
      mesh=mesh,
      scratch_types=[
          pltpu.VMEM((LANES,), jnp.float32),
          [pltpu.VMEM((CHUNK,), jnp.float32) for _ in range(NBUF)],
          [pltpu.VMEM((CHUNK,), jnp.float32) for _ in range(NBUF)],
          [pltpu.SemaphoreType.DMA for _ in range(NBUF)],
          [pltpu.SemaphoreType.DMA for _ in range(NBUF)],
      ],
  )(x_flat, pts_pad)


def _tc_body(sc_ref, pts_ref, x_ref, o_ref):
  del sc_ref
  pts = pts_ref[...]
  d = pts[:, 1:] - pts[:, :-1]
  c = pts[:, 0]
  for k in range(1, N_REG):
    c = c - float(k) * d[:, k]
  xn = x_ref[...] * _INV_LEN + _SHIFT
  acc = c[:, None, None] + d[:, 0][:, None, None] * jnp.minimum(xn, 1.0)
  for k in range(1, N_REG - 1):
    acc += d[:, k][:, None, None] * jnp.clip(xn, float(k), float(k + 1))
  acc += d[:, N_REG - 1][:, None, None] * jnp.maximum(xn, float(N_REG - 1))
  o_ref[...] = acc


def _pwlu_tc(sc3, points, x3):
  grid = ((NROWS - S_SC) // RB,)
  return pl.pallas_call(
      _tc_body,
      grid=grid,
      in_specs=[
          pl.BlockSpec(memory_space=pl.ANY),
          pl.BlockSpec((RB, N_PTS), lambda i: ((SB + i) % (N_CH // RB), 0)),
          pl.BlockSpec((RB,) + LANES_2D, lambda i: (SB + i, 0, 0)),
      ],
      out_specs=pl.BlockSpec((RB,) + LANES_2D, lambda i: (SB + i, 0, 0)),
      out_shape=jax.ShapeDtypeStruct((NROWS,) + LANES_2D, jnp.float32),
      input_output_aliases={0: 0},
  )(sc3, points, x3)


@jax.jit
def _pwlu(x, points):
  pts_pad = jnp.zeros((N_CH, LANES), jnp.float32).at[:, :N_PTS].set(points)
  sc_out = _pwlu_sc(x.reshape(-1), pts_pad)
  sc3 = sc_out.reshape((NROWS,) + LANES_2D)
  x3 = x.reshape((NROWS,) + LANES_2D)
  return _pwlu_tc(sc3, points, x3)


def kernel(x, points):
  return _pwlu(x, points).reshape(x.shape)


# speedup vs baseline: 1.2631x; 1.2631x over previous
import jax
import jax.numpy as jnp
from jax import lax
from jax.experimental import pallas as pl
from jax.experimental.pallas import tpu as pltpu
from jax.experimental.pallas import tpu_sc as plsc

N_CH = 192
N_PTS = 7
BOUND = 2.7
N_REG = N_PTS - 1
ROW = 224 * 224
NROWS = 4 * N_CH
LANES = 16

S_SC = 256
RB = 8
SB = S_SC // RB

NW = 32
NBUF = 4
CPR = 4
CHUNK = ROW // CPR
LANES_2D = (392, 128)

_INV_LEN = float(N_REG) / (2.0 * BOUND)
_SHIFT = BOUND * _INV_LEN


def _take16(vec, idx):
  return vec.at[idx].get(mode="promise_in_bounds")


def _sc_body(x_hbm, pts_hbm, out_hbm, pts_row, inbufs, outbufs, isems, osems):
  wid = lax.axis_index("s") * 2 + lax.axis_index("c")
  rows_per_w = S_SC // NW
  chunks_per_w = rows_per_w * CPR
  base_chunk = wid * chunks_per_w

  def start_in(g, b):
    pltpu.async_copy(x_hbm.at[pl.ds((base_chunk + g) * CHUNK, CHUNK)],
                     inbufs[b], isems[b])

  def wait_in(b):
    pltpu.make_async_copy(x_hbm.at[pl.ds(0, CHUNK)], inbufs[b],
                          isems[b]).wait()

  def start_out(g, b):
    pltpu.async_copy(outbufs[b],
                     out_hbm.at[pl.ds((base_chunk + g) * CHUNK, CHUNK)],
                     osems[b])

  def wait_out(b):
    pltpu.make_async_copy(outbufs[b], out_hbm.at[pl.ds(0, CHUNK)],
                          osems[b]).wait()

  for b in range(NBUF):
    start_in(b, b)

  lanes = lax.iota(jnp.int32, LANES)
  shift_idx = jnp.minimum(lanes + 1, LANES - 1)
  lanes_f = lanes.astype(jnp.float32)

  def row_body(j, carry):
    row = wid * rows_per_w + j
    ch = lax.rem(row, N_CH)
    pltpu.sync_copy(pts_hbm.at[ch], pts_row)
    p = pts_row[...]
    d = _take16(p, shift_idx) - p
    a = p - lanes_f * d

    for bb in range(CPR):
      g = j * CPR + bb
      b = bb
      wait_in(b)

      @pl.when(g >= NBUF)
      def _():
        wait_out(b)

      @plsc.parallel_loop(0, CHUNK, step=LANES, unroll=16)
      def _(off):
        xv = inbufs[b][pl.ds(off, LANES)]
        xn = xv * _INV_LEN + _SHIFT
        ri = jnp.minimum(jnp.maximum(xn.astype(jnp.int32), 0), N_REG - 1)
        outbufs[b][pl.ds(off, LANES)] = (
            _take16(a, ri) + xn * _take16(d, ri))

      start_out(g, b)

      @pl.when(g < chunks_per_w - NBUF)
      def _():
        start_in(g + NBUF, b)

    return carry

  lax.fori_loop(0, rows_per_w, row_body, 0)
  for b in range(NBUF):
    wait_out(b)


def _pwlu_sc(x_flat, pts_pad):
  mesh = plsc.VectorSubcoreMesh(core_axis_name="c", subcore_axis_name="s")
  return pl.kernel(
      _sc_body,
      out_type=jax.ShapeDtypeStruct((NROWS * ROW,), jnp.float32),
      mesh=mesh,
      scratch_types=[
          pltpu.VMEM((LANES,), jnp.float32),
          [pltpu.VMEM((CHUNK,), jnp.float32) for _ in range(NBUF)],
          [pltpu.VMEM((CHUNK,), jnp.float32) for _ in range(NBUF)],
          [pltpu.SemaphoreType.DMA for _ in range(NBUF)],
          [pltpu.SemaphoreType.DMA for _ in range(NBUF)],
      ],
  )(x_flat, pts_pad)


def _tc_body(sc_ref, pts_ref, x_ref, o_ref):
  del sc_ref
  pts = pts_ref[...]
  d = pts[:, 1:] - pts[:, :-1]
  c = pts[:, 0]
  for k in range(1, N_REG):
    c = c - float(k) * d[:, k]
  xn = x_ref[...] * _INV_LEN + _SHIFT
  acc = c[:, None, None] + d[:, 0][:, None, None] * jnp.minimum(xn, 1.0)
  for k in range(1, N_REG - 1):
    acc += d[:, k][:, None, None] * jnp.clip(xn, float(k), float(k + 1))
  acc += d[:, N_REG - 1][:, None, None] * jnp.maximum(xn, float(N_REG - 1))
  o_ref[...] = acc


def _tc_body_solo(pts_ref, x_ref, o_ref):
  _tc_body(None, pts_ref, x_ref, o_ref)


def _pwlu_tc_solo(points, x3):
  grid = (NROWS // RB,)
  return pl.pallas_call(
      _tc_body_solo,
      grid=grid,
      in_specs=[
          pl.BlockSpec((RB, N_PTS), lambda i: (i % (N_CH // RB), 0)),
          pl.BlockSpec((RB,) + LANES_2D, lambda i: (i, 0, 0)),
      ],
      out_specs=pl.BlockSpec((RB,) + LANES_2D, lambda i: (i, 0, 0)),
      out_shape=jax.ShapeDtypeStruct((NROWS,) + LANES_2D, jnp.float32),
  )(points, x3)


@jax.jit
def _pwlu(x, points):
  x3 = x.reshape((NROWS,) + LANES_2D)
  return _pwlu_tc_solo(points, x3)


def kernel(x, points):
  return _pwlu(x, points).reshape(x.shape)
